# baseline (speedup 1.0000x reference)
import functools

import jax
import jax.numpy as jnp
from jax import lax
from jax.experimental import pallas as pl
from jax.experimental.pallas import tpu as pltpu
from jax.experimental.pallas import tpu_sc as plsc

N = 10000
E = 160000
D = 256
H = 128
NPAD = 10240
ROWS_PER_TEC = NPAD // 16
CHUNK = 176
NR = 58
EPT = CHUNK * NR
E_PAD = EPT * 16
WCHUNK = 80


def _mm_body(lo_ref, hi_ref, w_ref, out_ref):
    xblk = jnp.concatenate([lo_ref[...], hi_ref[...]], axis=1)
    out_ref[...] = jnp.dot(xblk, w_ref[...], preferred_element_type=jnp.float32)


def _matmul(lo, hi, w):
    blk = 1000
    grid = (N // blk, 2)
    return pl.pallas_call(
        _mm_body,
        grid=grid,
        in_specs=[
            pl.BlockSpec((blk, H), lambda i, j: (i, 0)),
            pl.BlockSpec((blk, H), lambda i, j: (i, 0)),
            pl.BlockSpec((D, H), lambda i, j: (0, j)),
        ],
        out_specs=pl.BlockSpec((blk, H), lambda i, j: (j * (N // blk) + i, 0)),
        out_shape=jax.ShapeDtypeStruct((2 * N, H), jnp.float32),
    )(lo, hi, w)


def _zero16():
    return jnp.zeros((16,), jnp.float32)


_GDN = lax.GatherDimensionNumbers(
    offset_dims=(), collapsed_slice_dims=(0,), start_index_map=(0,))


def _splat(vec, lane):
    idx = jnp.full((16, 1), lane, jnp.int32)
    return lax.gather(vec, idx, _GDN, slice_sizes=(1,),
                      mode=lax.GatherScatterMode.PROMISE_IN_BOUNDS)


def _prop_body(apply_relu,
               table, src_hbm, dst_hbm, log_hbm, bias_hbm, out_hbm,
               rows0, rows1, src0, src1, dst0, dst1, expl0, expl1,
               bias_v, denom_v,
               isem0, isem1, gsem0, gsem1, ssem0, ssem1,
               acc_sh, denom_sh):
    c = lax.axis_index("c")
    s = lax.axis_index("s")
    rowsb = (rows0, rows1)
    srcb = (src0, src1)
    dstb = (dst0, dst1)
    explb = (expl0, expl1)
    isem = (isem0, isem1)
    gsem = (gsem0, gsem1)
    ssem = (ssem0, ssem1)

    rowbase = s * ROWS_PER_TEC
    ebase = s * EPT
    coff = c * N

    def zrow(i, _):
        for j in range(H // 16):
            rows0[i, pl.ds(16 * j, 16)] = _zero16()
        return 0

    lax.fori_loop(0, CHUNK, zrow, 0)

    def zden(i, _):
        denom_v[pl.ds(16 * i, 16)] = _zero16()
        return 0

    lax.fori_loop(0, ROWS_PER_TEC // 16, zden, 0)

    for q in range(4):
        pltpu.sync_copy(rows0.at[pl.ds(0, 160)],
                        acc_sh.at[pl.ds(rowbase + q * 160, 160)])
    pltpu.sync_copy(denom_v, denom_sh.at[pl.ds(rowbase, ROWS_PER_TEC)])
    pltpu.sync_copy(bias_hbm, bias_v)

    plsc.subcore_barrier()

    def issue_idx(r, p):
        b = ebase + r * CHUNK
        di = pltpu.async_copy(src_hbm.at[pl.ds(b, CHUNK)], srcb[p], isem[p])
        dj = pltpu.async_copy(dst_hbm.at[pl.ds(b, CHUNK)], dstb[p], isem[p])
        dk = pltpu.async_copy(log_hbm.at[pl.ds(b, CHUNK)], explb[p], isem[p])
        return (di, dj, dk)

    def prep(p):
        coffv = jnp.full((16,), coff, jnp.int32)

        def pstep(m, _):
            srcb[p][pl.ds(16 * m, 16)] = srcb[p][pl.ds(16 * m, 16)] + coffv
            explb[p][pl.ds(16 * m, 16)] = jnp.exp(explb[p][pl.ds(16 * m, 16)])
            return 0

        lax.fori_loop(0, CHUNK // 16, pstep, 0)

    def scale(p):
        def sstep(m, _):
            ev = explb[p][pl.ds(16 * m, 16)]
            for jj in range(16):
                spl = _splat(ev, jj)
                row = rowsb[p].at[16 * m + jj]
                for j in range(H // 16):
                    row[pl.ds(16 * j, 16)] = row[pl.ds(16 * j, 16)] * spl
            return 0

        lax.fori_loop(0, CHUNK // 16, sstep, 0)

    def gather(p):
        return pltpu.async_copy(table.at[srcb[p]], rowsb[p], gsem[p])

    def scatter(p):
        da = pltpu.async_copy(rowsb[p], acc_sh.at[dstb[p]], ssem[p], add=True)
        db = pltpu.async_copy(explb[p], denom_sh.at[dstb[p]], ssem[p], add=True)
        return (da, db)

    def pair(i, _):
        r0 = 2 * i
        i0 = issue_idx(r0, 0)
        i1 = issue_idx(r0 + 1, 1)
        for d in i0:
            d.wait()
        prep(0)
        g0 = gather(0)
        for d in i1:
            d.wait()
        prep(1)
        g1 = gather(1)
        g0.wait()
        scale(0)
        s0 = scatter(0)
        g1.wait()
        scale(1)
        s1 = scatter(1)
        for d in s0 + s1:
            d.wait()
        return 0

    lax.fori_loop(0, NR // 2, pair, 0)

    plsc.subcore_barrier()

    pltpu.sync_copy(denom_sh.at[pl.ds(rowbase, ROWS_PER_TEC)], denom_v)

    def wchunk(cc, _):
        base = rowbase + cc * WCHUNK

        @pl.when(base < N)
        def _():
            pltpu.sync_copy(acc_sh.at[pl.ds(base, WCHUNK)],
                            rows0.at[pl.ds(0, WCHUNK)])

            def node(m, _):
                dv = denom_v[pl.ds(cc * WCHUNK + 16 * m, 16)] + jnp.full(
                    (16,), 1e-16, jnp.float32)
                for jj in range(16):
                    dspl = _splat(dv, jj)
                    row = rows0.at[16 * m + jj]
                    for j in range(H // 16):
                        v = (row[pl.ds(16 * j, 16)] / dspl
                             + bias_v[pl.ds(c * H + 16 * j, 16)])
                        if apply_relu:
                            v = jnp.maximum(v, jnp.zeros((16,), jnp.float32))
                        row[pl.ds(16 * j, 16)] = v
                return 0

            lax.fori_loop(0, WCHUNK // 16, node, 0)
            pltpu.sync_copy(rows0.at[pl.ds(0, WCHUNK)],
                            out_hbm.at[c, pl.ds(base, WCHUNK)])

        return 0

    lax.fori_loop(0, ROWS_PER_TEC // WCHUNK, wchunk, 0)


def _prop(table, src, dst, logits, bias, apply_relu):
    mesh = plsc.VectorSubcoreMesh(core_axis_name="c", subcore_axis_name="s")
    kfn = pl.kernel(
        functools.partial(_prop_body, apply_relu),
        out_type=jax.ShapeDtypeStruct((2, N, H), jnp.float32),
        mesh=mesh,
        scratch_types=[
            pltpu.VMEM((CHUNK, H), jnp.float32),
            pltpu.VMEM((CHUNK, H), jnp.float32),
            pltpu.VMEM((CHUNK,), jnp.int32),
            pltpu.VMEM((CHUNK,), jnp.int32),
            pltpu.VMEM((CHUNK,), jnp.int32),
            pltpu.VMEM((CHUNK,), jnp.int32),
            pltpu.VMEM((CHUNK,), jnp.float32),
            pltpu.VMEM((CHUNK,), jnp.float32),
            pltpu.VMEM((2 * H,), jnp.float32),
            pltpu.VMEM((ROWS_PER_TEC,), jnp.float32),
            pltpu.SemaphoreType.DMA,
            pltpu.SemaphoreType.DMA,
            pltpu.SemaphoreType.DMA,
            pltpu.SemaphoreType.DMA,
            pltpu.SemaphoreType.DMA,
            pltpu.SemaphoreType.DMA,
            pltpu.VMEM_SHARED((NPAD, H), jnp.float32),
            pltpu.VMEM_SHARED((NPAD,), jnp.float32),
        ],
        name="gcn_prop",
    )
    return kfn(table, src, dst, logits, bias)


def kernel(x, edge_index, edge_weight_logits, W1, b1, W2, b2):
    pad = E_PAD - E
    src = jnp.concatenate([edge_index[0], jnp.zeros((pad,), jnp.int32)])
    dst = jnp.concatenate([edge_index[1], jnp.zeros((pad,), jnp.int32)])
    logits = jnp.concatenate(
        [edge_weight_logits, jnp.full((pad,), -1e30, jnp.float32)])
    h1 = _matmul(x[:, :H], x[:, H:], W1)
    o1 = _prop(h1, src, dst, logits, b1, apply_relu=True)
    h2 = _matmul(o1[0], o1[1], W2)
    o2 = _prop(h2, src, dst, logits, b2, apply_relu=False)
    return jnp.concatenate([o2[0], o2[1]], axis=1)[None]


# speedup vs baseline: 1.3197x; 1.3197x over previous
import functools

import jax
import jax.numpy as jnp
from jax import lax
from jax.experimental import pallas as pl
from jax.experimental.pallas import tpu as pltpu
from jax.experimental.pallas import tpu_sc as plsc

N = 10000
E = 160000
D = 256
H = 128
NPAD = 10240
ROWS_PER_TEC = NPAD // 16
CHUNK = 336
NR = 30
EPT = CHUNK * NR
E_PAD = EPT * 16
WCHUNK = 80


def _mm_body(lo_ref, hi_ref, w_ref, out_ref):
    xblk = jnp.concatenate([lo_ref[...], hi_ref[...]], axis=1)
    out_ref[...] = jnp.dot(xblk, w_ref[...], preferred_element_type=jnp.float32)


def _matmul(lo, hi, w):
    blk = 1000
    grid = (N // blk, 2)
    return pl.pallas_call(
        _mm_body,
        grid=grid,
        in_specs=[
            pl.BlockSpec((blk, H), lambda i, j: (i, 0)),
            pl.BlockSpec((blk, H), lambda i, j: (i, 0)),
            pl.BlockSpec((D, H), lambda i, j: (0, j)),
        ],
        out_specs=pl.BlockSpec((blk, H), lambda i, j: (j * (N // blk) + i, 0)),
        out_shape=jax.ShapeDtypeStruct((2 * N, H), jnp.float32),
    )(lo, hi, w)


def _zero16():
    return jnp.zeros((16,), jnp.float32)


_GDN = lax.GatherDimensionNumbers(
    offset_dims=(), collapsed_slice_dims=(0,), start_index_map=(0,))


def _splat(vec, lane):
    idx = jnp.full((16, 1), lane, jnp.int32)
    return lax.gather(vec, idx, _GDN, slice_sizes=(1,),
                      mode=lax.GatherScatterMode.PROMISE_IN_BOUNDS)


def _prop_body(apply_relu,
               table, src_hbm, dst_hbm, log_hbm, bias_hbm, out_hbm,
               rowsb, src0, src1, dst0, dst1, expl0, expl1, bias_v, denom_v,
               isem0, isem1, gsem, ssem,
               acc_sh, denom_sh):
    c = lax.axis_index("c")
    s = lax.axis_index("s")
    srcb = (src0, src1)
    dstb = (dst0, dst1)
    explb = (expl0, expl1)
    isem = (isem0, isem1)

    rowbase = s * ROWS_PER_TEC
    coff = c * N

    def zrow(i, _):
        for j in range(H // 16):
            rowsb[i, pl.ds(16 * j, 16)] = _zero16()
        return 0

    lax.fori_loop(0, CHUNK, zrow, 0)

    def zden(i, _):
        denom_v[pl.ds(16 * i, 16)] = _zero16()
        return 0

    lax.fori_loop(0, ROWS_PER_TEC // 16, zden, 0)

    for q in range(4):
        pltpu.sync_copy(rowsb.at[pl.ds(0, 160)],
                        acc_sh.at[pl.ds(rowbase + q * 160, 160)])
    pltpu.sync_copy(denom_v, denom_sh.at[pl.ds(rowbase, ROWS_PER_TEC)])
    pltpu.sync_copy(bias_hbm.at[pl.ds(c * H, H)], bias_v)

    plsc.subcore_barrier()

    def issue_idx(r, p):
        b = s * EPT + r * CHUNK
        pltpu.async_copy(src_hbm.at[pl.ds(b, CHUNK)], srcb[p], isem[p])
        pltpu.async_copy(dst_hbm.at[pl.ds(b, CHUNK)], dstb[p], isem[p])
        pltpu.async_copy(log_hbm.at[pl.ds(b, CHUNK)], explb[p], isem[p])

    def wait_idx(r, p):
        b = s * EPT + r * CHUNK
        pltpu.make_async_copy(src_hbm.at[pl.ds(b, CHUNK)], srcb[p], isem[p]).wait()
        pltpu.make_async_copy(dst_hbm.at[pl.ds(b, CHUNK)], dstb[p], isem[p]).wait()
        pltpu.make_async_copy(log_hbm.at[pl.ds(b, CHUNK)], explb[p], isem[p]).wait()

    def prep(p):
        coffv = jnp.full((16,), coff, jnp.int32)

        def pstep(m, _):
            srcb[p][pl.ds(16 * m, 16)] = srcb[p][pl.ds(16 * m, 16)] + coffv
            explb[p][pl.ds(16 * m, 16)] = jnp.exp(explb[p][pl.ds(16 * m, 16)])
            return 0

        lax.fori_loop(0, CHUNK // 16, pstep, 0)

    def scale(p):
        def sstep(m, _):
            ev = explb[p][pl.ds(16 * m, 16)]
            for jj in range(16):
                spl = _splat(ev, jj)
                row = rowsb.at[16 * m + jj]
                for j in range(H // 16):
                    row[pl.ds(16 * j, 16)] = row[pl.ds(16 * j, 16)] * spl
            return 0

        lax.fori_loop(0, CHUNK // 16, sstep, 0)

    def drain_scatter(q):
        pltpu.make_async_copy(rowsb, acc_sh.at[dstb[q]], ssem).wait()
        pltpu.make_async_copy(explb[q], denom_sh.at[dstb[q]], ssem).wait()

    def round_fn(r, p, q):
        wait_idx(r, p)
        prep(p)

        @pl.when(r >= 1)
        def _():
            drain_scatter(q)

        g = pltpu.async_copy(table.at[srcb[p]], rowsb, gsem)

        @pl.when(r + 1 < NR)
        def _():
            issue_idx(r + 1, q)

        g.wait()
        scale(p)
        pltpu.async_copy(rowsb, acc_sh.at[dstb[p]], ssem, add=True)
        pltpu.async_copy(explb[p], denom_sh.at[dstb[p]], ssem, add=True)

    issue_idx(0, 0)

    def pair(i, _):
        round_fn(2 * i, 0, 1)
        round_fn(2 * i + 1, 1, 0)
        return 0

    lax.fori_loop(0, NR // 2, pair, 0)
    drain_scatter(1)

    plsc.subcore_barrier()

    pltpu.sync_copy(denom_sh.at[pl.ds(rowbase, ROWS_PER_TEC)], denom_v)

    def wchunk(cc, _):
        base = rowbase + cc * WCHUNK

        @pl.when(base < N)
        def _():
            pltpu.sync_copy(acc_sh.at[pl.ds(base, WCHUNK)],
                            rowsb.at[pl.ds(0, WCHUNK)])

            def node(m, _):
                dv = denom_v[pl.ds(cc * WCHUNK + 16 * m, 16)] + jnp.full(
                    (16,), 1e-16, jnp.float32)
                for jj in range(16):
                    dspl = _splat(dv, jj)
                    row = rowsb.at[16 * m + jj]
                    for j in range(H // 16):
                        v = (row[pl.ds(16 * j, 16)] / dspl
                             + bias_v[pl.ds(16 * j, 16)])
                        if apply_relu:
                            v = jnp.maximum(v, jnp.zeros((16,), jnp.float32))
                        row[pl.ds(16 * j, 16)] = v
                return 0

            lax.fori_loop(0, WCHUNK // 16, node, 0)
            pltpu.sync_copy(rowsb.at[pl.ds(0, WCHUNK)],
                            out_hbm.at[c, pl.ds(base, WCHUNK)])

        return 0

    lax.fori_loop(0, ROWS_PER_TEC // WCHUNK, wchunk, 0)


def _prop(table, src, dst, logits, bias, apply_relu):
    mesh = plsc.VectorSubcoreMesh(core_axis_name="c", subcore_axis_name="s")
    kfn = pl.kernel(
        functools.partial(_prop_body, apply_relu),
        out_type=jax.ShapeDtypeStruct((2, N, H), jnp.float32),
        mesh=mesh,
        scratch_types=[
            pltpu.VMEM((CHUNK, H), jnp.float32),
            pltpu.VMEM((CHUNK,), jnp.int32),
            pltpu.VMEM((CHUNK,), jnp.int32),
            pltpu.VMEM((CHUNK,), jnp.int32),
            pltpu.VMEM((CHUNK,), jnp.int32),
            pltpu.VMEM((CHUNK,), jnp.float32),
            pltpu.VMEM((CHUNK,), jnp.float32),
            pltpu.VMEM((H,), jnp.float32),
            pltpu.VMEM((ROWS_PER_TEC,), jnp.float32),
            pltpu.SemaphoreType.DMA,
            pltpu.SemaphoreType.DMA,
            pltpu.SemaphoreType.DMA,
            pltpu.SemaphoreType.DMA,
            pltpu.VMEM_SHARED((NPAD, H), jnp.float32),
            pltpu.VMEM_SHARED((NPAD,), jnp.float32),
        ],
        name="gcn_prop",
    )
    return kfn(table, src, dst, logits, bias)


def kernel(x, edge_index, edge_weight_logits, W1, b1, W2, b2):
    pad = E_PAD - E
    src = jnp.concatenate([edge_index[0], jnp.zeros((pad,), jnp.int32)])
    dst = jnp.concatenate([edge_index[1], jnp.zeros((pad,), jnp.int32)])
    logits = jnp.concatenate(
        [edge_weight_logits, jnp.full((pad,), -1e30, jnp.float32)])
    h1 = _matmul(x[:, :H], x[:, H:], W1)
    o1 = _prop(h1, src, dst, logits, b1, apply_relu=True)
    h2 = _matmul(o1[0], o1[1], W2)
    o2 = _prop(h2, src, dst, logits, b2, apply_relu=False)
    return jnp.concatenate([o2[0], o2[1]], axis=1)[None]
